# baseline (speedup 1.0000x reference)
import jax
import jax.numpy as jnp
from jax import lax
from jax.experimental import pallas as pl
from jax.experimental.pallas import tpu as pltpu
from jax.experimental.pallas import tpu_sc as plsc

NUM_ROWS_TBL = 100000
BATCH_N = 16384
LR = 200
LS = 50
NC = 2
NS = 16
NW = NC * NS
ROWS_PER_W = BATCH_N // NW
GROUPS = ROWS_PER_W // 16
GPC = 2
CHUNKS = GROUPS // GPC
TBL_WORDS = NUM_ROWS_TBL
SYN_OFF_W = NUM_ROWS_TBL // 2

HI_MASK = jnp.int32(-65536)


def _decode_acc(tbl_v, col, word_base, acc):
    word = word_base + lax.shift_right_logical(col, 1)
    bits = plsc.load_gather(tbl_v, [word])
    odd = col & 1
    val = jnp.where(odd == 1, bits & HI_MASK, lax.shift_left(bits, 16))
    return acc + plsc.bitcast(val, jnp.float32)


def _sc_body(rule_2d, syn_2d, tbl_hbm, gb_hbm, out_hbm,
             tbl_v, ridx_v0, ridx_v1, sidx_v0, sidx_v1, acc_v, gb_v,
             sem0, sem1, semt):
    wid = lax.axis_index("s") * NC + lax.axis_index("c")
    base = wid * ROWS_PER_W

    lane = jnp.arange(16, dtype=jnp.int32)
    zero16 = jnp.zeros((16,), jnp.float32)
    sems = (sem0, sem1)
    rbufs = (ridx_v0, ridx_v1)
    sbufs = (sidx_v0, sidx_v1)

    def chunk_dma(c, slot):
        row0 = pl.multiple_of(base + c * (GPC * 16), 8)
        dr = pltpu.async_copy(rule_2d.at[pl.ds(row0, GPC * 16), :],
                              rbufs[slot], sems[slot])
        ds = pltpu.async_copy(syn_2d.at[pl.ds(row0, GPC * 16), :],
                              sbufs[slot], sems[slot])
        return (dr, ds)

    tdma = pltpu.async_copy(tbl_hbm, tbl_v, semt)
    pending = chunk_dma(0, 0)
    pltpu.sync_copy(gb_hbm, gb_v)
    tdma.wait()

    UR = 8
    US = 5

    def rule_step_for(buf, k):
        rows = lane + k * 16

        def rule_step(i, accs):
            j0 = i * UR
            out = []
            for u in range(UR):
                col = plsc.load_gather(buf, [rows, jnp.full((16,), j0 + u,
                                                            jnp.int32)])
                out.append(_decode_acc(tbl_v, col, 0, accs[u]))
            return tuple(out)
        return rule_step

    def syn_step_for(buf, k):
        rows = lane + k * 16

        def syn_step(i, accs):
            j0 = i * US
            out = []
            for u in range(US):
                col = plsc.load_gather(buf, [rows, jnp.full((16,), j0 + u,
                                                            jnp.int32)])
                out.append(_decode_acc(tbl_v, col, SYN_OFF_W, accs[u]))
            return tuple(out)
        return syn_step

    gamma = gb_v[pl.ds(0, 16)]
    bias = gb_v[pl.ds(16, 16)]

    for c in range(CHUNKS):
        pending[0].wait()
        pending[1].wait()
        if c + 1 < CHUNKS:
            pending = chunk_dma(c + 1, (c + 1) % 2)
        for k in range(GPC):
            g = c * GPC + k
            accs = lax.fori_loop(0, LR // UR,
                                 rule_step_for(rbufs[c % 2], k),
                                 (zero16,) * UR)
            sacc = lax.fori_loop(0, LS // US,
                                 syn_step_for(sbufs[c % 2], k),
                                 accs[:US])
            acc = sacc[0]
            for u in range(1, US):
                acc = acc + sacc[u]
            for u in range(US, UR):
                acc = acc + accs[u]
            score = gamma * acc + bias
            score = jnp.minimum(jnp.maximum(score, 0.0), 30.0)
            acc_v[pl.ds(g * 16, 16)] = 1.0 - jnp.exp(-score)

    pltpu.sync_copy(acc_v, out_hbm.at[pl.ds(base, ROWS_PER_W)])


@jax.jit
def _surprisal_sc(rule_2d, syn_2d, tbl, gb):
    mesh = plsc.VectorSubcoreMesh(core_axis_name="c", subcore_axis_name="s",
                                  num_cores=NC, num_subcores=NS)
    return pl.kernel(
        _sc_body,
        out_type=jax.ShapeDtypeStruct((BATCH_N,), jnp.float32),
        mesh=mesh,
        compiler_params=pltpu.CompilerParams(needs_layout_passes=False),
        scratch_types=[
            pltpu.VMEM((TBL_WORDS,), jnp.int32),
            pltpu.VMEM((GPC * 16, LR), jnp.int32),
            pltpu.VMEM((GPC * 16, LR), jnp.int32),
            pltpu.VMEM((GPC * 16, LS), jnp.int32),
            pltpu.VMEM((GPC * 16, LS), jnp.int32),
            pltpu.VMEM((ROWS_PER_W,), jnp.float32),
            pltpu.VMEM((32,), jnp.float32),
            pltpu.SemaphoreType.DMA,
            pltpu.SemaphoreType.DMA,
            pltpu.SemaphoreType.DMA,
        ],
    )(rule_2d, syn_2d, tbl, gb)


def kernel(rule_idx, synergy_idx, rules_w, synergy_w, bias, gamma):
    rsq = jnp.square(rules_w[:NUM_ROWS_TBL, 0]).astype(jnp.bfloat16)
    ssq = jnp.square(synergy_w[:NUM_ROWS_TBL, 0]).astype(jnp.bfloat16)
    tbl = jax.lax.bitcast_convert_type(
        jnp.concatenate([rsq, ssq]).reshape(TBL_WORDS, 2), jnp.int32)
    gb = jnp.concatenate([jnp.broadcast_to(gamma, (16,)),
                          jnp.broadcast_to(bias, (16,))])
    return _surprisal_sc(rule_idx.astype(jnp.int32),
                         synergy_idx.astype(jnp.int32), tbl, gb)


# speedup vs baseline: 1.8397x; 1.8397x over previous
import jax
import jax.numpy as jnp
from jax import lax
from jax.experimental import pallas as pl
from jax.experimental.pallas import tpu as pltpu
from jax.experimental.pallas import tpu_sc as plsc

NUM_ROWS_TBL = 100000
BATCH_N = 16384
LR = 200
LS = 50
NC = 2
NS = 16
NW = NC * NS
ROWS_PER_W = BATCH_N // NW
GROUPS = ROWS_PER_W // 16
GPC = 4
CHUNKS = GROUPS // GPC
RCH = GPC * 16 * LR
SGPC = 2
SCHUNKS = GROUPS // SGPC
SCH = SGPC * 16 * LS


def _sc_body(rule_flat, syn_flat, rw_hbm, sw_hbm, gb_hbm, out_hbm,
             table_v, ridx_v0, ridx_v1, sidx_v0, sidx_v1, acc_v, gb_v,
             sem0, sem1, semt):
    wid = lax.axis_index("s") * NC + lax.axis_index("c")
    base = wid * ROWS_PER_W

    lane = jnp.arange(16, dtype=jnp.int32)
    zero16 = jnp.zeros((16,), jnp.float32)
    sems = (sem0, sem1)
    rbufs = (ridx_v0, ridx_v1)
    sbufs = (sidx_v0, sidx_v1)
    lane_r = lane * LR
    lane_s = lane * LS

    def rule_dma(c, buf_slot):
        off = pl.multiple_of(base * LR + c * RCH, 8)
        return pltpu.async_copy(rule_flat.at[pl.ds(off, RCH)],
                                rbufs[buf_slot], sems[buf_slot])

    def syn_dma(c, buf_slot):
        off = pl.multiple_of(base * LS + c * SCH, 8)
        return pltpu.async_copy(syn_flat.at[pl.ds(off, SCH)],
                                sbufs[buf_slot], sems[buf_slot])

    pending = rule_dma(0, 0)
    tdma = pltpu.async_copy(rw_hbm.at[pl.ds(0, NUM_ROWS_TBL)], table_v, semt)
    pltpu.sync_copy(gb_hbm, gb_v)
    tdma.wait()

    for c in range(CHUNKS):
        pending.wait()
        if c + 1 < CHUNKS:
            pending = rule_dma(c + 1, (c + 1) % 2)
        for k in range(GPC):
            buf = rbufs[c % 2].at[pl.ds(k * 16 * LR, 16 * LR)]

            @plsc.parallel_loop(0, LR, unroll=8, carry=zero16)
            def _racc(j, acc, buf=buf):
                col = plsc.load_gather(buf, [lane_r + j])
                w = plsc.load_gather(table_v, [col])
                return acc + w * w

            acc_v[pl.ds((c * GPC + k) * 16, 16)] = _racc

    pending = syn_dma(0, 0)
    pltpu.sync_copy(sw_hbm.at[pl.ds(0, NUM_ROWS_TBL)], table_v)

    gamma = gb_v[pl.ds(0, 16)]
    bias = gb_v[pl.ds(16, 16)]

    for c in range(SCHUNKS):
        pending.wait()
        if c + 1 < SCHUNKS:
            pending = syn_dma(c + 1, (c + 1) % 2)
        for k in range(SGPC):
            g = c * SGPC + k
            buf = sbufs[c % 2].at[pl.ds(k * 16 * LS, 16 * LS)]

            @plsc.parallel_loop(0, LS, unroll=5,
                                carry=acc_v[pl.ds(g * 16, 16)])
            def _sacc(j, acc, buf=buf):
                col = plsc.load_gather(buf, [lane_s + j])
                w = plsc.load_gather(table_v, [col])
                return acc + w * w

            score = gamma * _sacc + bias
            score = jnp.minimum(jnp.maximum(score, 0.0), 30.0)
            acc_v[pl.ds(g * 16, 16)] = 1.0 - jnp.exp(-score)

    pltpu.sync_copy(acc_v, out_hbm.at[pl.ds(base, ROWS_PER_W)])


@jax.jit
def _surprisal_sc(rule_flat, syn_flat, rw, sw, gb):
    mesh = plsc.VectorSubcoreMesh(core_axis_name="c", subcore_axis_name="s",
                                  num_cores=NC, num_subcores=NS)
    return pl.kernel(
        _sc_body,
        out_type=jax.ShapeDtypeStruct((BATCH_N,), jnp.float32),
        mesh=mesh,
        compiler_params=pltpu.CompilerParams(needs_layout_passes=False),
        scratch_types=[
            pltpu.VMEM((NUM_ROWS_TBL,), jnp.float32),
            pltpu.VMEM((RCH,), jnp.int32),
            pltpu.VMEM((RCH,), jnp.int32),
            pltpu.VMEM((SCH,), jnp.int32),
            pltpu.VMEM((SCH,), jnp.int32),
            pltpu.VMEM((ROWS_PER_W,), jnp.float32),
            pltpu.VMEM((32,), jnp.float32),
            pltpu.SemaphoreType.DMA,
            pltpu.SemaphoreType.DMA,
            pltpu.SemaphoreType.DMA,
        ],
    )(rule_flat, syn_flat, rw, sw, gb)


def kernel(rule_idx, synergy_idx, rules_w, synergy_w, bias, gamma):
    rule_flat = rule_idx.astype(jnp.int32).reshape(-1)
    syn_flat = synergy_idx.astype(jnp.int32).reshape(-1)
    rw = rules_w.reshape(-1)
    sw = synergy_w.reshape(-1)
    gb = jnp.concatenate([jnp.broadcast_to(gamma, (16,)),
                          jnp.broadcast_to(bias, (16,))])
    return _surprisal_sc(rule_flat, syn_flat, rw, sw, gb)


# speedup vs baseline: 1.9662x; 1.0688x over previous
import jax
import jax.numpy as jnp
from jax import lax
from jax.experimental import pallas as pl
from jax.experimental.pallas import tpu as pltpu
from jax.experimental.pallas import tpu_sc as plsc

NUM_ROWS_TBL = 100000
BATCH_N = 16384
LR = 200
LS = 50
NC = 2
NS = 16
NW = NC * NS
ROWS_PER_W = BATCH_N // NW
GROUPS = ROWS_PER_W // 16
GPC = 2
CHUNKS = GROUPS // GPC
SGPC = 2
SCHUNKS = GROUPS // SGPC
SCH = SGPC * 16 * LS


def _sc_body(rule_2d, syn_flat, rw_hbm, sw_hbm, gb_hbm, out_hbm,
             table_v, r2d_v0, r2d_v1, rflat_v, sidx_v0, sidx_v1,
             acc_v, gb_v, sem0, sem1, semt):
    wid = lax.axis_index("s") * NC + lax.axis_index("c")
    base = wid * ROWS_PER_W

    lane = jnp.arange(16, dtype=jnp.int32)
    zero16 = jnp.zeros((16,), jnp.float32)
    sems = (sem0, sem1)
    r2ds = (r2d_v0, r2d_v1)
    sbufs = (sidx_v0, sidx_v1)
    lane_r = lane * LR
    lane_s = lane * LS

    def rule_dma(c, slot):
        row0 = pl.multiple_of(base + c * (GPC * 16), 8)
        return pltpu.async_copy(rule_2d.at[pl.ds(row0, GPC * 16), :],
                                r2ds[slot], sems[slot])

    def syn_dma(c, slot):
        off = pl.multiple_of(base * LS + c * SCH, 8)
        return pltpu.async_copy(syn_flat.at[pl.ds(off, SCH)],
                                sbufs[slot], sems[slot])

    pending = rule_dma(0, 0)
    tdma = pltpu.async_copy(rw_hbm.at[pl.ds(0, NUM_ROWS_TBL)], table_v, semt)
    pltpu.sync_copy(gb_hbm, gb_v)
    tdma.wait()

    j0s = list(range(0, LR - 15, 16))
    if LR % 16:
        j0s.append(LR - 16)

    def repack(src2d):
        @plsc.parallel_loop(0, GPC * 16, unroll=2)
        def _row(r):
            rfull = jnp.full((16,), r, jnp.int32)
            for j0 in j0s:
                v = plsc.load_gather(src2d, [rfull, lane + j0])
                rflat_v[pl.ds(r * LR + j0, 16)] = v

    for c in range(CHUNKS):
        pending.wait()
        if c + 1 < CHUNKS:
            pending = rule_dma(c + 1, (c + 1) % 2)
        repack(r2ds[c % 2])
        for k in range(GPC):
            g = c * GPC + k
            rbuf = rflat_v.at[pl.ds(k * 16 * LR, 16 * LR)]

            @plsc.parallel_loop(0, LR, unroll=8, carry=zero16)
            def _racc(j, acc, rbuf=rbuf):
                col = plsc.load_gather(rbuf, [lane_r + j])
                w = plsc.load_gather(table_v, [col])
                return acc + w * w

            acc_v[pl.ds(g * 16, 16)] = _racc

    pending = syn_dma(0, 0)
    pltpu.sync_copy(sw_hbm.at[pl.ds(0, NUM_ROWS_TBL)], table_v)

    gamma = gb_v[pl.ds(0, 16)]
    bias = gb_v[pl.ds(16, 16)]

    for c in range(SCHUNKS):
        pending.wait()
        if c + 1 < SCHUNKS:
            pending = syn_dma(c + 1, (c + 1) % 2)
        for k in range(SGPC):
            g = c * SGPC + k
            sbuf = sbufs[c % 2].at[pl.ds(k * 16 * LS, 16 * LS)]

            @plsc.parallel_loop(0, LS, unroll=5,
                                carry=acc_v[pl.ds(g * 16, 16)])
            def _sacc(j, acc, sbuf=sbuf):
                col = plsc.load_gather(sbuf, [lane_s + j])
                w = plsc.load_gather(table_v, [col])
                return acc + w * w

            score = gamma * _sacc + bias
            score = jnp.minimum(jnp.maximum(score, 0.0), 30.0)
            acc_v[pl.ds(g * 16, 16)] = 1.0 - jnp.exp(-score)

    pltpu.sync_copy(acc_v, out_hbm.at[pl.ds(base, ROWS_PER_W)])


@jax.jit
def _surprisal_sc(rule_2d, syn_flat, rw, sw, gb):
    mesh = plsc.VectorSubcoreMesh(core_axis_name="c", subcore_axis_name="s",
                                  num_cores=NC, num_subcores=NS)
    return pl.kernel(
        _sc_body,
        out_type=jax.ShapeDtypeStruct((BATCH_N,), jnp.float32),
        mesh=mesh,
        compiler_params=pltpu.CompilerParams(needs_layout_passes=False),
        scratch_types=[
            pltpu.VMEM((NUM_ROWS_TBL,), jnp.float32),
            pltpu.VMEM((GPC * 16, LR), jnp.int32),
            pltpu.VMEM((GPC * 16, LR), jnp.int32),
            pltpu.VMEM((GPC * 16 * LR,), jnp.int32),
            pltpu.VMEM((SCH,), jnp.int32),
            pltpu.VMEM((SCH,), jnp.int32),
            pltpu.VMEM((ROWS_PER_W,), jnp.float32),
            pltpu.VMEM((32,), jnp.float32),
            pltpu.SemaphoreType.DMA,
            pltpu.SemaphoreType.DMA,
            pltpu.SemaphoreType.DMA,
        ],
    )(rule_2d, syn_flat, rw, sw, gb)


def kernel(rule_idx, synergy_idx, rules_w, synergy_w, bias, gamma):
    syn_flat = synergy_idx.astype(jnp.int32).reshape(-1)
    gb = jnp.concatenate([jnp.broadcast_to(gamma, (16,)),
                          jnp.broadcast_to(bias, (16,))])
    return _surprisal_sc(rule_idx.astype(jnp.int32), syn_flat,
                         rules_w.reshape(-1), synergy_w.reshape(-1), gb)
